# Initial kernel scaffold; baseline (speedup 1.0000x reference)
import jax
import jax.numpy as jnp
from jax.experimental import pallas as pl


def kernel(x, W, b):
    raise NotImplementedError("write your pallas kernel here")


# speedup vs baseline: 20.0884x; 20.0884x over previous
import functools
import numpy as np
import jax
import jax.numpy as jnp
from jax import lax
from jax.experimental import pallas as pl
from jax.experimental.pallas import tpu as pltpu
from jax.experimental.pallas import tpu_sc as plsc

B = 128
NB = 202
NCH = 200
P = 25
NPIX = P * P
PADN = 640
NCHUNK = PADN // 16
CB = 50
NWORK = 32
BPW = B // NWORK

_rng = np.random.RandomState(0)
_size = int(NPIX * 0.08)
_idx_all = np.stack(
    [_rng.choice(np.arange(NPIX), replace=False, size=_size) for _ in range(B)],
    axis=0)
_forced_np = np.zeros((B, PADN), dtype=np.int32)
_forced_np[np.arange(B)[:, None], _idx_all] = 1
_FORCED = _forced_np

_masks_np = np.zeros((9, PADN), dtype=np.float32)
for _k in range(9):
    _dy, _dx = _k // 3 - 1, _k % 3 - 1
    for _p in range(NPIX):
        _y, _x = _p // P, _p % P
        if 0 <= _y + _dy < P and 0 <= _x + _dx < P:
            _masks_np[_k, _p] = 1.0
_MASKS = _masks_np


def _sc_body(x_rows, forced, xo, sp_v, forced_v, m_v, rz_v, onepos_v, g_v,
             in_v, out_v):
    wid = lax.axis_index("s") * 2 + lax.axis_index("c")
    lane = lax.iota(jnp.int32, 16)

    def batch_body(bi, _):
        b = wid * BPW + bi
        pltpu.sync_copy(x_rows.at[b * NB + NCH], sp_v.at[pl.ds(0, NPIX)])
        pltpu.sync_copy(forced.at[b], forced_v)
        cvec = plsc.load_gather(sp_v, [jnp.zeros((16,), jnp.int32) + 312])
        c_is0 = cvec == 0.0
        last = jnp.zeros((16,), jnp.int32) + 15

        def chunk_a(j, carry):
            c1, c0 = carry
            base = j * 16
            pv = lane + base
            spj = sp_v[pl.ds(base, 16)]
            fj = forced_v[pl.ds(base, 16)]
            inb = pv < NPIX
            mbool = ((spj == cvec) | c_is0 | (fj > 0)) & inb
            mj = mbool.astype(jnp.int32)
            zj = (inb & (~mbool)).astype(jnp.int32)
            cum1 = plsc.cumsum(mj) + c1
            cum0 = plsc.cumsum(zj) + c0
            m_v[pl.ds(base, 16)] = mj
            rz_v[pl.ds(base, 16)] = cum0 - zj
            plsc.store_scatter(onepos_v, [cum1 - mj], pv, mask=mbool)
            return (jnp.take_along_axis(cum1, last, axis=0),
                    jnp.take_along_axis(cum0, last, axis=0))

        n1vec, _unused = lax.fori_loop(
            0, NCHUNK, chunk_a,
            (jnp.zeros((16,), jnp.int32), jnp.zeros((16,), jnp.int32)))

        def chunk_b(j, _):
            base = j * 16
            pv = lane + base
            mj = m_v[pl.ds(base, 16)]
            t = lax.rem(rz_v[pl.ds(base, 16)], n1vec)
            src = plsc.load_gather(onepos_v, [t])
            g_v[pl.ds(base, 16)] = jnp.where(mj > 0, pv, src)
            return 0

        lax.fori_loop(0, NCHUNK, chunk_b, 0)

        def cb_body(cb, _):
            row0 = b * NB + cb * CB
            pltpu.sync_copy(x_rows.at[pl.ds(row0, CB)], in_v)

            def k_body(k, _):
                ksplat = jnp.zeros((16,), jnp.int32) + k

                def j_body(j, _):
                    base = j * 16
                    gj = g_v[pl.ds(base, 16)]
                    vals = plsc.load_gather(in_v, [ksplat, gj])
                    out_v[pl.ds(k * PADN + base, 16)] = vals
                    return 0

                lax.fori_loop(0, NCHUNK, j_body, 0)
                return 0

            lax.fori_loop(0, CB, k_body, 0)
            pltpu.sync_copy(out_v, xo.at[b, pl.ds(cb * CB * PADN, CB * PADN)])
            return 0

        lax.fori_loop(0, NCH // CB, cb_body, 0)
        return 0

    lax.fori_loop(0, BPW, batch_body, 0)


@functools.cache
def _sc_gather():
    mesh = plsc.VectorSubcoreMesh(core_axis_name="c", subcore_axis_name="s")
    return pl.kernel(
        _sc_body,
        mesh=mesh,
        compiler_params=pltpu.CompilerParams(
            use_tc_tiling_on_sc=False, needs_layout_passes=False),
        out_type=jax.ShapeDtypeStruct((B, NCH * PADN), jnp.float32),
        scratch_types=[
            pltpu.VMEM((PADN,), jnp.float32),
            pltpu.VMEM((PADN,), jnp.int32),
            pltpu.VMEM((PADN,), jnp.int32),
            pltpu.VMEM((PADN,), jnp.int32),
            pltpu.VMEM((PADN,), jnp.int32),
            pltpu.VMEM((PADN,), jnp.int32),
            pltpu.VMEM((CB, NPIX), jnp.float32),
            pltpu.VMEM((CB * PADN,), jnp.float32),
        ],
    )


def _conv_body(xo_ref, wt_ref, mask_ref, b_ref, out_ref):
    x2 = xo_ref[0]
    acc = jnp.zeros((32, PADN), jnp.float32)
    for k in range(9):
        dy, dx = k // 3 - 1, k % 3 - 1
        s = dy * P + dx
        if s > 0:
            sh = jnp.concatenate([x2[:, s:], x2[:, :s]], axis=1)
        elif s < 0:
            sh = jnp.concatenate([x2[:, s:], x2[:, :PADN + s]], axis=1)
        else:
            sh = x2
        sh = sh * mask_ref[k][None, :]
        acc = acc + lax.dot_general(
            wt_ref[k], sh, (((1,), (0,)), ((), ())),
            preferred_element_type=jnp.float32)
    out_ref[0] = (acc + b_ref[...])[:, :NPIX]


_conv = pl.pallas_call(
    _conv_body,
    grid=(B,),
    in_specs=[
        pl.BlockSpec((1, NCH, PADN), lambda i: (i, 0, 0)),
        pl.BlockSpec((9, 32, NCH), lambda i: (0, 0, 0)),
        pl.BlockSpec((9, PADN), lambda i: (0, 0)),
        pl.BlockSpec((32, 1), lambda i: (0, 0)),
    ],
    out_specs=pl.BlockSpec((1, 32, NPIX), lambda i: (i, 0, 0)),
    out_shape=jax.ShapeDtypeStruct((B, 32, NPIX), jnp.float32),
)


def kernel(x, W, b):
    xr = jnp.reshape(x, (B * NB, NPIX))
    xo = _sc_gather()(xr, jnp.asarray(_FORCED))
    xo3 = jnp.reshape(xo, (B, NCH, PADN))
    wt = jnp.reshape(jnp.transpose(W, (2, 3, 0, 1)), (9, 32, NCH))
    y = _conv(xo3, wt, jnp.asarray(_MASKS), jnp.reshape(b, (32, 1)))
    return jnp.reshape(y, (B, 32, P, P))


# speedup vs baseline: 31.7499x; 1.5805x over previous
import functools
import numpy as np
import jax
import jax.numpy as jnp
from jax import lax
from jax.experimental import pallas as pl
from jax.experimental.pallas import tpu as pltpu
from jax.experimental.pallas import tpu_sc as plsc

B = 128
NB = 202
NCH = 200
P = 25
NPIX = P * P
PADN = 640
NCHUNK = PADN // 16
CB = 40
NBLK = NCH // CB
NWORK = 32
BPW = B // NWORK

_rng = np.random.RandomState(0)
_size = int(NPIX * 0.08)
_idx_all = np.stack(
    [_rng.choice(np.arange(NPIX), replace=False, size=_size) for _ in range(B)],
    axis=0)
_forced_np = np.zeros((B, PADN), dtype=np.int32)
_forced_np[np.arange(B)[:, None], _idx_all] = 1
_FORCED = _forced_np

_masks_np = np.zeros((9, PADN), dtype=np.float32)
for _k in range(9):
    _dy, _dx = _k // 3 - 1, _k % 3 - 1
    for _p in range(NPIX):
        _y, _x = _p // P, _p % P
        if 0 <= _y + _dy < P and 0 <= _x + _dx < P:
            _masks_np[_k, _p] = 1.0
_MASKS = _masks_np


def _sc_body(x_rows, forced, xo, sp_v, forced_v, m_v, rz_v, onepos_v, g_v,
             in_v0, in_v1, out_v0, out_v1, in_sem, out_sem):
    wid = lax.axis_index("s") * 2 + lax.axis_index("c")
    lane = lax.iota(jnp.int32, 16)

    def batch_body(bi, _):
        b = wid * BPW + bi
        pltpu.sync_copy(x_rows.at[b * NB + NCH], sp_v.at[pl.ds(0, NPIX)])
        pltpu.sync_copy(forced.at[b], forced_v)
        cvec = plsc.load_gather(sp_v, [jnp.zeros((16,), jnp.int32) + 312])
        c_is0 = cvec == 0.0
        last = jnp.zeros((16,), jnp.int32) + 15

        def chunk_a(j, carry):
            c1, c0 = carry
            base = j * 16
            pv = lane + base
            spj = sp_v[pl.ds(base, 16)]
            fj = forced_v[pl.ds(base, 16)]
            inb = pv < NPIX
            mbool = ((spj == cvec) | c_is0 | (fj > 0)) & inb
            mj = mbool.astype(jnp.int32)
            zj = (inb & (~mbool)).astype(jnp.int32)
            cum1 = plsc.cumsum(mj) + c1
            cum0 = plsc.cumsum(zj) + c0
            m_v[pl.ds(base, 16)] = mj
            rz_v[pl.ds(base, 16)] = cum0 - zj
            plsc.store_scatter(onepos_v, [cum1 - mj], pv, mask=mbool)
            return (jnp.take_along_axis(cum1, last, axis=0),
                    jnp.take_along_axis(cum0, last, axis=0))

        n1vec, _unused = lax.fori_loop(
            0, NCHUNK, chunk_a,
            (jnp.zeros((16,), jnp.int32), jnp.zeros((16,), jnp.int32)))

        @plsc.parallel_loop(0, NCHUNK, unroll=8)
        def chunk_b(j):
            base = j * 16
            pv = lane + base
            mj = m_v[pl.ds(base, 16)]
            t = lax.rem(rz_v[pl.ds(base, 16)], n1vec)
            src = plsc.load_gather(onepos_v, [t])
            g_v[pl.ds(base, 16)] = jnp.where(mj > 0, pv, src)

        in_bufs = (in_v0, in_v1)
        out_bufs = (out_v0, out_v1)

        def start_in(cb, buf):
            return pltpu.async_copy(
                x_rows.at[pl.ds(b * NB + cb * CB, CB)], buf, in_sem)

        in_h = {0: start_in(0, in_bufs[0])}
        out_h = {}
        for cb in range(NBLK):
            ib = in_bufs[cb % 2]
            ob = out_bufs[cb % 2]
            in_h[cb % 2].wait()
            if cb + 1 < NBLK:
                in_h[(cb + 1) % 2] = start_in(cb + 1, in_bufs[(cb + 1) % 2])
            if cb % 2 in out_h:
                out_h[cb % 2].wait()

            @plsc.parallel_loop(0, CB * NCHUNK, unroll=8)
            def gather_t(t, ib=ib, ob=ob):
                k = t // NCHUNK
                base = (t - k * NCHUNK) * 16
                gj = g_v[pl.ds(base, 16)]
                vals = plsc.load_gather(
                    ib, [jnp.zeros((16,), jnp.int32) + k, gj])
                ob[pl.ds(k * PADN + base, 16)] = vals

            out_h[cb % 2] = pltpu.async_copy(
                ob, xo.at[b, pl.ds(cb * CB * PADN, CB * PADN)], out_sem)
        out_h[(NBLK - 1) % 2].wait()
        out_h[(NBLK - 2) % 2].wait()
        return 0

    lax.fori_loop(0, BPW, batch_body, 0)


@functools.cache
def _sc_gather():
    mesh = plsc.VectorSubcoreMesh(core_axis_name="c", subcore_axis_name="s")
    return pl.kernel(
        _sc_body,
        mesh=mesh,
        compiler_params=pltpu.CompilerParams(
            use_tc_tiling_on_sc=False, needs_layout_passes=False),
        out_type=jax.ShapeDtypeStruct((B, NCH * PADN), jnp.float32),
        scratch_types=[
            pltpu.VMEM((PADN,), jnp.float32),
            pltpu.VMEM((PADN,), jnp.int32),
            pltpu.VMEM((PADN,), jnp.int32),
            pltpu.VMEM((PADN,), jnp.int32),
            pltpu.VMEM((PADN,), jnp.int32),
            pltpu.VMEM((PADN,), jnp.int32),
            pltpu.VMEM((CB, NPIX), jnp.float32),
            pltpu.VMEM((CB, NPIX), jnp.float32),
            pltpu.VMEM((CB * PADN,), jnp.float32),
            pltpu.VMEM((CB * PADN,), jnp.float32),
            pltpu.SemaphoreType.DMA,
            pltpu.SemaphoreType.DMA,
        ],
    )


def _conv_body(xo_ref, wt_ref, mask_ref, b_ref, out_ref):
    x2 = xo_ref[0]
    acc = jnp.zeros((32, PADN), jnp.float32)
    for k in range(9):
        dy, dx = k // 3 - 1, k % 3 - 1
        s = dy * P + dx
        if s > 0:
            sh = jnp.concatenate([x2[:, s:], x2[:, :s]], axis=1)
        elif s < 0:
            sh = jnp.concatenate([x2[:, s:], x2[:, :PADN + s]], axis=1)
        else:
            sh = x2
        sh = sh * mask_ref[k][None, :]
        acc = acc + lax.dot_general(
            wt_ref[k], sh, (((1,), (0,)), ((), ())),
            preferred_element_type=jnp.float32)
    out_ref[0] = (acc + b_ref[...])[:, :NPIX]


_conv = pl.pallas_call(
    _conv_body,
    grid=(B,),
    in_specs=[
        pl.BlockSpec((1, NCH, PADN), lambda i: (i, 0, 0)),
        pl.BlockSpec((9, 32, NCH), lambda i: (0, 0, 0)),
        pl.BlockSpec((9, PADN), lambda i: (0, 0)),
        pl.BlockSpec((32, 1), lambda i: (0, 0)),
    ],
    out_specs=pl.BlockSpec((1, 32, NPIX), lambda i: (i, 0, 0)),
    out_shape=jax.ShapeDtypeStruct((B, 32, NPIX), jnp.float32),
)


def kernel(x, W, b):
    xr = jnp.reshape(x, (B * NB, NPIX))
    xo = _sc_gather()(xr, jnp.asarray(_FORCED))
    xo3 = jnp.reshape(xo, (B, NCH, PADN))
    wt = jnp.reshape(jnp.transpose(W, (2, 3, 0, 1)), (9, 32, NCH))
    y = _conv(xo3, wt, jnp.asarray(_MASKS), jnp.reshape(b, (32, 1)))
    return jnp.reshape(y, (B, 32, P, P))


# speedup vs baseline: 33.9454x; 1.0692x over previous
import functools
import numpy as np
import jax
import jax.numpy as jnp
from jax import lax
from jax.experimental import pallas as pl
from jax.experimental.pallas import tpu as pltpu
from jax.experimental.pallas import tpu_sc as plsc

B = 128
NB = 202
NCH = 200
P = 25
NPIX = P * P
PADN = 640
NCHUNK = PADN // 16
CB = 40
NBLK = NCH // CB
NWORK = 32
BPW = B // NWORK

_rng = np.random.RandomState(0)
_size = int(NPIX * 0.08)
_idx_all = np.stack(
    [_rng.choice(np.arange(NPIX), replace=False, size=_size) for _ in range(B)],
    axis=0)
_forced_np = np.zeros((B, PADN), dtype=np.int32)
_forced_np[np.arange(B)[:, None], _idx_all] = 1
_FORCED = _forced_np

_maskx_np = np.zeros((3, PADN), dtype=np.float32)
_masky_np = np.zeros((3, PADN), dtype=np.float32)
for _d in range(3):
    for _p in range(PADN):
        _y, _x = _p // P, _p % P
        if 0 <= (_p % P) + _d - 1 < P:
            _maskx_np[_d, _p] = 1.0
        if _p < NPIX and 0 <= (_p // P) + _d - 1 < P:
            _masky_np[_d, _p] = 1.0
_MASKX = _maskx_np
_MASKY = _masky_np


def _sc_body(x_rows, forced, xo, sp_v, forced_v, m_v, rz_v, onepos_v, g_v,
             in_v0, in_v1, out_v0, out_v1, in_sem, out_sem):
    wid = lax.axis_index("s") * 2 + lax.axis_index("c")
    lane = lax.iota(jnp.int32, 16)

    def batch_body(bi, _):
        b = wid * BPW + bi
        pltpu.sync_copy(x_rows.at[b * NB + NCH], sp_v.at[pl.ds(0, NPIX)])
        pltpu.sync_copy(forced.at[b], forced_v)
        cvec = plsc.load_gather(sp_v, [jnp.zeros((16,), jnp.int32) + 312])
        c_is0 = cvec == 0.0
        last = jnp.zeros((16,), jnp.int32) + 15

        def chunk_a(j, carry):
            c1, c0 = carry
            base = j * 16
            pv = lane + base
            spj = sp_v[pl.ds(base, 16)]
            fj = forced_v[pl.ds(base, 16)]
            inb = pv < NPIX
            mbool = ((spj == cvec) | c_is0 | (fj > 0)) & inb
            mj = mbool.astype(jnp.int32)
            zj = (inb & (~mbool)).astype(jnp.int32)
            cum1 = plsc.cumsum(mj) + c1
            cum0 = plsc.cumsum(zj) + c0
            m_v[pl.ds(base, 16)] = mj
            rz_v[pl.ds(base, 16)] = cum0 - zj
            plsc.store_scatter(onepos_v, [cum1 - mj], pv, mask=mbool)
            return (jnp.take_along_axis(cum1, last, axis=0),
                    jnp.take_along_axis(cum0, last, axis=0))

        n1vec, _unused = lax.fori_loop(
            0, NCHUNK, chunk_a,
            (jnp.zeros((16,), jnp.int32), jnp.zeros((16,), jnp.int32)))

        @plsc.parallel_loop(0, NCHUNK, unroll=8)
        def chunk_b(j):
            base = j * 16
            pv = lane + base
            mj = m_v[pl.ds(base, 16)]
            t = lax.rem(rz_v[pl.ds(base, 16)], n1vec)
            src = plsc.load_gather(onepos_v, [t])
            g_v[pl.ds(base, 16)] = jnp.where(mj > 0, pv, src)

        in_bufs = (in_v0, in_v1)
        out_bufs = (out_v0, out_v1)

        def start_in(cb, buf):
            return pltpu.async_copy(
                x_rows.at[pl.ds(b * NB + cb * CB, CB)], buf, in_sem)

        in_h = {0: start_in(0, in_bufs[0])}
        out_h = {}
        for cb in range(NBLK):
            ib = in_bufs[cb % 2]
            ob = out_bufs[cb % 2]
            in_h[cb % 2].wait()
            if cb + 1 < NBLK:
                in_h[(cb + 1) % 2] = start_in(cb + 1, in_bufs[(cb + 1) % 2])
            if cb % 2 in out_h:
                out_h[cb % 2].wait()

            @plsc.parallel_loop(0, CB * NCHUNK, unroll=8)
            def gather_t(t, ib=ib, ob=ob):
                k = t // NCHUNK
                base = (t - k * NCHUNK) * 16
                gj = g_v[pl.ds(base, 16)]
                vals = plsc.load_gather(
                    ib, [jnp.zeros((16,), jnp.int32) + k, gj])
                ob[pl.ds(k * PADN + base, 16)] = vals

            out_h[cb % 2] = pltpu.async_copy(
                ob, xo.at[b, pl.ds(cb * CB * PADN, CB * PADN)], out_sem)
        out_h[(NBLK - 1) % 2].wait()
        out_h[(NBLK - 2) % 2].wait()
        return 0

    lax.fori_loop(0, BPW, batch_body, 0)


@functools.cache
def _sc_gather():
    mesh = plsc.VectorSubcoreMesh(core_axis_name="c", subcore_axis_name="s")
    return pl.kernel(
        _sc_body,
        mesh=mesh,
        compiler_params=pltpu.CompilerParams(
            use_tc_tiling_on_sc=False, needs_layout_passes=False),
        out_type=jax.ShapeDtypeStruct((B, NCH * PADN), jnp.float32),
        scratch_types=[
            pltpu.VMEM((PADN,), jnp.float32),
            pltpu.VMEM((PADN,), jnp.int32),
            pltpu.VMEM((PADN,), jnp.int32),
            pltpu.VMEM((PADN,), jnp.int32),
            pltpu.VMEM((PADN,), jnp.int32),
            pltpu.VMEM((PADN,), jnp.int32),
            pltpu.VMEM((CB, NPIX), jnp.float32),
            pltpu.VMEM((CB, NPIX), jnp.float32),
            pltpu.VMEM((CB * PADN,), jnp.float32),
            pltpu.VMEM((CB * PADN,), jnp.float32),
            pltpu.SemaphoreType.DMA,
            pltpu.SemaphoreType.DMA,
        ],
    )


def _rot(a, s):
    if s > 0:
        return jnp.concatenate([a[:, s:], a[:, :s]], axis=1)
    if s < 0:
        return jnp.concatenate([a[:, s:], a[:, :PADN + s]], axis=1)
    return a


def _conv_body(xo_ref, wt_ref, maskx_ref, masky_ref, b_ref, out_ref):
    x2 = xo_ref[0]
    us = [_rot(x2, dx) * maskx_ref[dx + 1][None, :] for dx in (-1, 0, 1)]
    acc = b_ref[...].astype(jnp.float32) * jnp.ones((32, PADN), jnp.float32)
    for dy in (-1, 0, 1):
        part = jnp.zeros((32, PADN), jnp.float32)
        for dx in (-1, 0, 1):
            k = (dy + 1) * 3 + (dx + 1)
            part = part + lax.dot_general(
                wt_ref[k], us[dx + 1], (((1,), (0,)), ((), ())),
                preferred_element_type=jnp.float32)
        acc = acc + _rot(part, dy * P) * masky_ref[dy + 1][None, :]
    out_ref[0] = acc[:, :NPIX]


_conv = pl.pallas_call(
    _conv_body,
    grid=(B,),
    in_specs=[
        pl.BlockSpec((1, NCH, PADN), lambda i: (i, 0, 0)),
        pl.BlockSpec((9, 32, NCH), lambda i: (0, 0, 0)),
        pl.BlockSpec((3, PADN), lambda i: (0, 0)),
        pl.BlockSpec((3, PADN), lambda i: (0, 0)),
        pl.BlockSpec((32, 1), lambda i: (0, 0)),
    ],
    out_specs=pl.BlockSpec((1, 32, NPIX), lambda i: (i, 0, 0)),
    out_shape=jax.ShapeDtypeStruct((B, 32, NPIX), jnp.float32),
)


def kernel(x, W, b):
    xr = jnp.reshape(x, (B * NB, NPIX))
    xo = _sc_gather()(xr, jnp.asarray(_FORCED))
    xo3 = jnp.reshape(xo, (B, NCH, PADN))
    wt = jnp.reshape(jnp.transpose(W, (2, 3, 0, 1)), (9, 32, NCH))
    y = _conv(xo3, wt, jnp.asarray(_MASKX), jnp.asarray(_MASKY),
              jnp.reshape(b, (32, 1)))
    return jnp.reshape(y, (B, 32, P, P))
